# Initial kernel scaffold; baseline (speedup 1.0000x reference)
import functools

import jax
import jax.numpy as jnp
from jax import lax
from jax.experimental import pallas as pl

LEVELS = [10000, 2500, 625, 157, 40]
SEQ = 9


def _conv_body(g_ref, w_ref, b_ref, o_ref, *, elu):
    g = g_ref[0]
    w = w_ref[...]
    y = lax.dot_general(g, w, (((1,), (1,)), ((), ())),
                        preferred_element_type=jnp.float32)
    y = y + b_ref[...][None, :]
    if elu:
        y = jnp.where(y > 0, y, jnp.expm1(y))
    o_ref[0] = y


def _conv(g, w, b, elu, blk):
    B_, N, K = g.shape
    cout = w.shape[0]
    npad = ((N + blk - 1) // blk) * blk
    if npad != N:
        g = jnp.pad(g, ((0, 0), (0, npad - N), (0, 0)))
    out = pl.pallas_call(
        functools.partial(_conv_body, elu=elu),
        grid=(B_, npad // blk),
        in_specs=[
            pl.BlockSpec((1, blk, K), lambda i, j: (i, j, 0)),
            pl.BlockSpec((cout, K), lambda i, j: (0, 0)),
            pl.BlockSpec((cout,), lambda i, j: (0,)),
        ],
        out_specs=pl.BlockSpec((1, blk, cout), lambda i, j: (i, j, 0)),
        out_shape=jax.ShapeDtypeStruct((B_, npad, cout), jnp.float32),
    )(g, w, b)
    return out[:, :N] if npad != N else out


def _pool_body(p0_ref, p1_ref, p2_ref, v0_ref, v1_ref, v2_ref, o_ref):
    o_ref[0] = (p0_ref[0] * v0_ref[...] + p1_ref[0] * v1_ref[...]
                + p2_ref[0] * v2_ref[...])


def _pool_reduce(p0, p1, p2, v0, v1, v2, blk):
    B_, nd, c = p0.shape
    npad = ((nd + blk - 1) // blk) * blk
    if npad != nd:
        pad3 = ((0, 0), (0, npad - nd), (0, 0))
        p0 = jnp.pad(p0, pad3)
        p1 = jnp.pad(p1, pad3)
        p2 = jnp.pad(p2, pad3)
        v0 = jnp.pad(v0, ((0, npad - nd), (0, 0)))
        v1 = jnp.pad(v1, ((0, npad - nd), (0, 0)))
        v2 = jnp.pad(v2, ((0, npad - nd), (0, 0)))
    out = pl.pallas_call(
        _pool_body,
        grid=(B_, npad // blk),
        in_specs=[
            pl.BlockSpec((1, blk, c), lambda i, j: (i, j, 0)),
            pl.BlockSpec((1, blk, c), lambda i, j: (i, j, 0)),
            pl.BlockSpec((1, blk, c), lambda i, j: (i, j, 0)),
            pl.BlockSpec((blk, 1), lambda i, j: (j, 0)),
            pl.BlockSpec((blk, 1), lambda i, j: (j, 0)),
            pl.BlockSpec((blk, 1), lambda i, j: (j, 0)),
        ],
        out_specs=pl.BlockSpec((1, blk, c), lambda i, j: (i, j, 0)),
        out_shape=jax.ShapeDtypeStruct((B_, npad, c), jnp.float32),
    )(p0, p1, p2, v0, v1, v2)
    return out[:, :nd] if npad != nd else out


def _mid_body(h_ref, wen_ref, ben_ref, eps_ref, wde_ref, bde_ref,
              mu_ref, lv_ref, h2_ref):
    h = h_ref[...]
    y = lax.dot_general(h, wen_ref[...], (((1,), (1,)), ((), ())),
                        preferred_element_type=jnp.float32)
    y = y + ben_ref[...][None, :]
    mu = y[:, :64]
    logvar = y[:, 64:]
    z = mu + eps_ref[...] * jnp.exp(0.5 * logvar)
    h2 = lax.dot_general(z, wde_ref[...], (((1,), (1,)), ((), ())),
                         preferred_element_type=jnp.float32)
    h2 = h2 + bde_ref[...][None, :]
    mu_ref[...] = mu
    lv_ref[...] = logvar
    h2_ref[...] = h2


def _mid(h, w_enfc, b_enfc, eps, w_defc, b_defc):
    B_ = h.shape[0]
    d = w_defc.shape[0]
    return pl.pallas_call(
        _mid_body,
        out_shape=(
            jax.ShapeDtypeStruct((B_, 64), jnp.float32),
            jax.ShapeDtypeStruct((B_, 64), jnp.float32),
            jax.ShapeDtypeStruct((B_, d), jnp.float32),
        ),
    )(h, w_enfc, b_enfc, eps, w_defc, b_defc)


def _gather_rows(h, idx):
    return jnp.take(h, idx, axis=1)


def _spiral_gather(h, si):
    b, n, c = h.shape
    g = _gather_rows(h, si.reshape(-1))
    return g.reshape(b, si.shape[0], si.shape[1] * c)


def _pool(h, cols, vals, n_dst, blk):
    cols3 = cols.reshape(n_dst, 3)
    vals3 = vals.reshape(n_dst, 3)
    p0 = _gather_rows(h, cols3[:, 0])
    p1 = _gather_rows(h, cols3[:, 1])
    p2 = _gather_rows(h, cols3[:, 2])
    return _pool_reduce(p0, p1, p2, vals3[:, 0:1], vals3[:, 1:2],
                        vals3[:, 2:3], blk)


def kernel(x, eps, si0, si1, si2, si3,
           d0_rows, d0_cols, d0_vals, d1_rows, d1_cols, d1_vals,
           d2_rows, d2_cols, d2_vals, d3_rows, d3_cols, d3_vals,
           u0_rows, u0_cols, u0_vals, u1_rows, u1_cols, u1_vals,
           u2_rows, u2_cols, u2_vals, u3_rows, u3_cols, u3_vals,
           W_en0, b_en0, W_en1, b_en1, W_en2, b_en2, W_en3, b_en3,
           W_enfc, b_enfc, W_defc, b_defc,
           W_de0, b_de0, W_de1, b_de1, W_de2, b_de2, W_de3, b_de3,
           W_out, b_out):
    si = [si0, si1, si2, si3]
    down = [(d0_cols, d0_vals), (d1_cols, d1_vals),
            (d2_cols, d2_vals), (d3_cols, d3_vals)]
    up = [(u0_cols, u0_vals), (u1_cols, u1_vals),
          (u2_cols, u2_vals), (u3_cols, u3_vals)]
    W_en = [W_en0, W_en1, W_en2, W_en3]
    b_en = [b_en0, b_en1, b_en2, b_en3]
    W_de = [W_de0, W_de1, W_de2, W_de3]
    b_de = [b_de0, b_de1, b_de2, b_de3]
    blks = [500, 500, 625, 157, 40]

    h = x
    for i in range(4):
        g = _spiral_gather(h, si[i])
        h = _conv(g, W_en[i], b_en[i], elu=True, blk=blks[i])
        cols, vals = down[i]
        h = _pool(h, cols, vals, LEVELS[i + 1], blks[i + 1])

    b_ = h.shape[0]
    mu, logvar, h2 = _mid(h.reshape(b_, -1), W_enfc, b_enfc, eps,
                          W_defc, b_defc)
    h = h2.reshape(b_, LEVELS[4], 64)

    for i in range(4):
        lvl = 3 - i
        cols, vals = up[lvl]
        h = _pool(h, cols, vals, LEVELS[lvl], blks[lvl])
        g = _spiral_gather(h, si[lvl])
        h = _conv(g, W_de[i], b_de[i], elu=True, blk=blks[lvl])

    g = _spiral_gather(h, si[0])
    re_x = _conv(g, W_out, b_out, elu=False, blk=blks[0])
    return (re_x, mu, logvar)


# speedup vs baseline: 1.1220x; 1.1220x over previous
import functools

import jax
import jax.numpy as jnp
from jax import lax
from jax.experimental import pallas as pl

LEVELS = [10000, 2500, 625, 157, 40]
SEQ = 9


def _conv_body(g_ref, w_ref, b_ref, o_ref, *, elu):
    g = g_ref[0]
    w = w_ref[...]
    y = lax.dot_general(g, w, (((1,), (1,)), ((), ())),
                        preferred_element_type=jnp.float32)
    y = y + b_ref[...][None, :]
    if elu:
        y = jnp.where(y > 0, y, jnp.exp(jnp.minimum(y, 0.0)) - 1.0)
    o_ref[0] = y


def _conv(g, w, b, elu, blk):
    B_, N, K = g.shape
    cout = w.shape[0]
    npad = ((N + blk - 1) // blk) * blk
    if npad != N:
        g = jnp.pad(g, ((0, 0), (0, npad - N), (0, 0)))
    out = pl.pallas_call(
        functools.partial(_conv_body, elu=elu),
        grid=(B_, npad // blk),
        in_specs=[
            pl.BlockSpec((1, blk, K), lambda i, j: (i, j, 0)),
            pl.BlockSpec((cout, K), lambda i, j: (0, 0)),
            pl.BlockSpec((cout,), lambda i, j: (0,)),
        ],
        out_specs=pl.BlockSpec((1, blk, cout), lambda i, j: (i, j, 0)),
        out_shape=jax.ShapeDtypeStruct((B_, npad, cout), jnp.float32),
    )(g, w, b)
    return out[:, :N] if npad != N else out


def _pool_body(p0_ref, p1_ref, p2_ref, v0_ref, v1_ref, v2_ref, o_ref):
    o_ref[0] = (p0_ref[0] * v0_ref[...] + p1_ref[0] * v1_ref[...]
                + p2_ref[0] * v2_ref[...])


def _pool_reduce(p0, p1, p2, v0, v1, v2, blk):
    B_, nd, c = p0.shape
    npad = ((nd + blk - 1) // blk) * blk
    if npad != nd:
        pad3 = ((0, 0), (0, npad - nd), (0, 0))
        p0 = jnp.pad(p0, pad3)
        p1 = jnp.pad(p1, pad3)
        p2 = jnp.pad(p2, pad3)
        v0 = jnp.pad(v0, ((0, npad - nd), (0, 0)))
        v1 = jnp.pad(v1, ((0, npad - nd), (0, 0)))
        v2 = jnp.pad(v2, ((0, npad - nd), (0, 0)))
    out = pl.pallas_call(
        _pool_body,
        grid=(B_, npad // blk),
        in_specs=[
            pl.BlockSpec((1, blk, c), lambda i, j: (i, j, 0)),
            pl.BlockSpec((1, blk, c), lambda i, j: (i, j, 0)),
            pl.BlockSpec((1, blk, c), lambda i, j: (i, j, 0)),
            pl.BlockSpec((blk, 1), lambda i, j: (j, 0)),
            pl.BlockSpec((blk, 1), lambda i, j: (j, 0)),
            pl.BlockSpec((blk, 1), lambda i, j: (j, 0)),
        ],
        out_specs=pl.BlockSpec((1, blk, c), lambda i, j: (i, j, 0)),
        out_shape=jax.ShapeDtypeStruct((B_, npad, c), jnp.float32),
    )(p0, p1, p2, v0, v1, v2)
    return out[:, :nd] if npad != nd else out


def _mid_body(h_ref, wen_ref, ben_ref, eps_ref, wde_ref, bde_ref,
              mu_ref, lv_ref, h2_ref):
    h = h_ref[...]
    y = lax.dot_general(h, wen_ref[...], (((1,), (1,)), ((), ())),
                        preferred_element_type=jnp.float32)
    y = y + ben_ref[...][None, :]
    mu = y[:, :64]
    logvar = y[:, 64:]
    z = mu + eps_ref[...] * jnp.exp(0.5 * logvar)
    h2 = lax.dot_general(z, wde_ref[...], (((1,), (1,)), ((), ())),
                         preferred_element_type=jnp.float32)
    h2 = h2 + bde_ref[...][None, :]
    mu_ref[...] = mu
    lv_ref[...] = logvar
    h2_ref[...] = h2


def _mid(h, w_enfc, b_enfc, eps, w_defc, b_defc):
    B_ = h.shape[0]
    d = w_defc.shape[0]
    return pl.pallas_call(
        _mid_body,
        out_shape=(
            jax.ShapeDtypeStruct((B_, 64), jnp.float32),
            jax.ShapeDtypeStruct((B_, 64), jnp.float32),
            jax.ShapeDtypeStruct((B_, d), jnp.float32),
        ),
    )(h, w_enfc, b_enfc, eps, w_defc, b_defc)


def _gather_rows(h, idx):
    return jnp.take(h, idx, axis=1)


def _spiral_gather(h, si):
    b, n, c = h.shape
    g = _gather_rows(h, si.reshape(-1))
    return g.reshape(b, si.shape[0], si.shape[1] * c)


def _pool(h, cols, vals, n_dst, blk):
    cols3 = cols.reshape(n_dst, 3)
    vals3 = vals.reshape(n_dst, 3)
    p0 = _gather_rows(h, cols3[:, 0])
    p1 = _gather_rows(h, cols3[:, 1])
    p2 = _gather_rows(h, cols3[:, 2])
    return _pool_reduce(p0, p1, p2, vals3[:, 0:1], vals3[:, 1:2],
                        vals3[:, 2:3], blk)


def kernel(x, eps, si0, si1, si2, si3,
           d0_rows, d0_cols, d0_vals, d1_rows, d1_cols, d1_vals,
           d2_rows, d2_cols, d2_vals, d3_rows, d3_cols, d3_vals,
           u0_rows, u0_cols, u0_vals, u1_rows, u1_cols, u1_vals,
           u2_rows, u2_cols, u2_vals, u3_rows, u3_cols, u3_vals,
           W_en0, b_en0, W_en1, b_en1, W_en2, b_en2, W_en3, b_en3,
           W_enfc, b_enfc, W_defc, b_defc,
           W_de0, b_de0, W_de1, b_de1, W_de2, b_de2, W_de3, b_de3,
           W_out, b_out):
    si = [si0, si1, si2, si3]
    down = [(d0_cols, d0_vals), (d1_cols, d1_vals),
            (d2_cols, d2_vals), (d3_cols, d3_vals)]
    up = [(u0_cols, u0_vals), (u1_cols, u1_vals),
          (u2_cols, u2_vals), (u3_cols, u3_vals)]
    W_en = [W_en0, W_en1, W_en2, W_en3]
    b_en = [b_en0, b_en1, b_en2, b_en3]
    W_de = [W_de0, W_de1, W_de2, W_de3]
    b_de = [b_de0, b_de1, b_de2, b_de3]
    blks = [512, 512, 640, 160, 40]

    h = x
    for i in range(4):
        g = _spiral_gather(h, si[i])
        h = _conv(g, W_en[i], b_en[i], elu=True, blk=blks[i])
        cols, vals = down[i]
        h = _pool(h, cols, vals, LEVELS[i + 1], blks[i + 1])

    b_ = h.shape[0]
    mu, logvar, h2 = _mid(h.reshape(b_, -1), W_enfc, b_enfc, eps,
                          W_defc, b_defc)
    h = h2.reshape(b_, LEVELS[4], 64)

    for i in range(4):
        lvl = 3 - i
        cols, vals = up[lvl]
        h = _pool(h, cols, vals, LEVELS[lvl], blks[lvl])
        g = _spiral_gather(h, si[lvl])
        h = _conv(g, W_de[i], b_de[i], elu=True, blk=blks[lvl])

    g = _spiral_gather(h, si[0])
    re_x = _conv(g, W_out, b_out, elu=False, blk=blks[0])
    return (re_x, mu, logvar)


# speedup vs baseline: 2.3152x; 2.0635x over previous
import functools

import jax
import jax.numpy as jnp
from jax import lax
from jax.experimental import pallas as pl
from jax.experimental.pallas import tpu as pltpu
from jax.experimental.pallas import tpu_sc as plsc

LEVELS = [10000, 2500, 625, 157, 40]
NPAD = [10240, 2560, 768, 256, 256]
SEQ = 9
_NC, _NS, _NW = 2, 16, 32
_MESH = dict(core_axis_name="c", subcore_axis_name="s")


def _sc_gather(h, idx):
    B_, _, C = h.shape
    M = idx.shape[0]
    m_per = M // _NW

    @functools.partial(
        pl.kernel,
        out_type=jax.ShapeDtypeStruct((B_, M, C), jnp.float32),
        mesh=plsc.VectorSubcoreMesh(**_MESH),
        compiler_params=pltpu.CompilerParams(use_tc_tiling_on_sc=False),
        scratch_types=[
            pltpu.VMEM((m_per,), jnp.int32),
            pltpu.VMEM((m_per, C), jnp.float32),
            pltpu.SemaphoreType.DMA,
        ],
    )
    def gk(h_hbm, idx_hbm, out_hbm, idx_v, rows_v, sem):
        wid = lax.axis_index("s") * _NC + lax.axis_index("c")
        base = wid * m_per
        pltpu.sync_copy(idx_hbm.at[pl.ds(base, m_per)], idx_v)

        def body(b, carry):
            cps = []
            off = 0
            while off < m_per:
                n = min(128, m_per - off)
                cps.append(pltpu.async_copy(
                    h_hbm.at[b].at[idx_v.at[pl.ds(off, n)]],
                    rows_v.at[pl.ds(off, n)], sem))
                off += n
            for cp in cps:
                cp.wait()
            pltpu.sync_copy(rows_v, out_hbm.at[b].at[pl.ds(base, m_per)])
            return carry

        lax.fori_loop(0, B_, body, 0)

    return gk(h, idx)


def _sc_pool(h, cols, vals, nd):
    B_, _, C = h.shape
    nd_per = nd // _NW
    e_per = 3 * nd_per

    @functools.partial(
        pl.kernel,
        out_type=jax.ShapeDtypeStruct((B_, nd, C), jnp.float32),
        mesh=plsc.VectorSubcoreMesh(**_MESH),
        compiler_params=pltpu.CompilerParams(use_tc_tiling_on_sc=False,
                                             needs_layout_passes=False),
        scratch_types=[
            pltpu.VMEM((e_per,), jnp.int32),
            pltpu.VMEM((e_per,), jnp.float32),
            pltpu.VMEM((e_per, C), jnp.float32),
            pltpu.VMEM((nd_per, C), jnp.float32),
            pltpu.SemaphoreType.DMA,
        ],
    )
    def pk(h_hbm, cols_hbm, vals_hbm, out_hbm, idx_v, w_v, g_v, o_v, sem):
        wid = lax.axis_index("s") * _NC + lax.axis_index("c")
        base = wid * nd_per
        pltpu.sync_copy(cols_hbm.at[pl.ds(3 * base, e_per)], idx_v)
        pltpu.sync_copy(vals_hbm.at[pl.ds(3 * base, e_per)], w_v)

        def body(b, carry):
            cps = []
            off = 0
            while off < e_per:
                n = min(128, e_per - off)
                cps.append(pltpu.async_copy(
                    h_hbm.at[b].at[idx_v.at[pl.ds(off, n)]],
                    g_v.at[pl.ds(off, n)], sem))
                off += n
            for cp in cps:
                cp.wait()

            def node(i, c2):
                w0 = plsc.load_gather(w_v, [jnp.full((16,), 3 * i, jnp.int32)])
                w1 = plsc.load_gather(w_v, [jnp.full((16,), 3 * i + 1, jnp.int32)])
                w2 = plsc.load_gather(w_v, [jnp.full((16,), 3 * i + 2, jnp.int32)])
                for k in range(C // 16):
                    s = pl.ds(k * 16, 16)
                    o_v[i, s] = (w0 * g_v[3 * i, s] + w1 * g_v[3 * i + 1, s]
                                 + w2 * g_v[3 * i + 2, s])
                return c2

            lax.fori_loop(0, nd_per, node, 0)
            pltpu.sync_copy(o_v, out_hbm.at[b].at[pl.ds(base, nd_per)])
            return carry

        lax.fori_loop(0, B_, body, 0)

    return pk(h, cols, vals)


def _conv_body(g_ref, w_ref, b_ref, o_ref, *, elu):
    g = g_ref[0]
    w = w_ref[...]
    y = lax.dot_general(g, w, (((1,), (1,)), ((), ())),
                        preferred_element_type=jnp.float32)
    y = y + b_ref[...][None, :]
    if elu:
        y = jnp.where(y > 0, y, jnp.exp(jnp.minimum(y, 0.0)) - 1.0)
    o_ref[0] = y


def _conv(g, w, b, elu, blk):
    B_, N, K = g.shape
    cout = w.shape[0]
    return pl.pallas_call(
        functools.partial(_conv_body, elu=elu),
        grid=(B_, N // blk),
        in_specs=[
            pl.BlockSpec((1, blk, K), lambda i, j: (i, j, 0)),
            pl.BlockSpec((cout, K), lambda i, j: (0, 0)),
            pl.BlockSpec((cout,), lambda i, j: (0,)),
        ],
        out_specs=pl.BlockSpec((1, blk, cout), lambda i, j: (i, j, 0)),
        out_shape=jax.ShapeDtypeStruct((B_, N, cout), jnp.float32),
    )(g, w, b)


def _mid_body(h_ref, wen_ref, ben_ref, eps_ref, wde_ref, bde_ref,
              mu_ref, lv_ref, h2_ref):
    h = h_ref[...]
    y = lax.dot_general(h, wen_ref[...], (((1,), (1,)), ((), ())),
                        preferred_element_type=jnp.float32)
    y = y + ben_ref[...][None, :]
    mu = y[:, :64]
    logvar = y[:, 64:]
    z = mu + eps_ref[...] * jnp.exp(0.5 * logvar)
    h2 = lax.dot_general(z, wde_ref[...], (((1,), (1,)), ((), ())),
                         preferred_element_type=jnp.float32)
    h2 = h2 + bde_ref[...][None, :]
    mu_ref[...] = mu
    lv_ref[...] = logvar
    h2_ref[...] = h2


def _mid(h, w_enfc, b_enfc, eps, w_defc, b_defc):
    B_ = h.shape[0]
    d = w_defc.shape[0]
    return pl.pallas_call(
        _mid_body,
        out_shape=(
            jax.ShapeDtypeStruct((B_, 64), jnp.float32),
            jax.ShapeDtypeStruct((B_, 64), jnp.float32),
            jax.ShapeDtypeStruct((B_, d), jnp.float32),
        ),
    )(h, w_enfc, b_enfc, eps, w_defc, b_defc)


def _spiral(h, si, npad_dst, w, b, elu, blk):
    c = h.shape[2]
    flat = si.reshape(-1)
    m = npad_dst * SEQ
    if flat.shape[0] != m:
        flat = jnp.pad(flat, (0, m - flat.shape[0]))
    g = _sc_gather(h, flat)
    g = g.reshape(h.shape[0], npad_dst, SEQ * c)
    return _conv(g, w, b, elu, blk)


def _pool(h, cols, vals, npad_dst):
    e = 3 * npad_dst
    if cols.shape[0] != e:
        cols = jnp.pad(cols, (0, e - cols.shape[0]))
        vals = jnp.pad(vals, (0, e - vals.shape[0]))
    return _sc_pool(h, cols, vals, npad_dst)


def kernel(x, eps, si0, si1, si2, si3,
           d0_rows, d0_cols, d0_vals, d1_rows, d1_cols, d1_vals,
           d2_rows, d2_cols, d2_vals, d3_rows, d3_cols, d3_vals,
           u0_rows, u0_cols, u0_vals, u1_rows, u1_cols, u1_vals,
           u2_rows, u2_cols, u2_vals, u3_rows, u3_cols, u3_vals,
           W_en0, b_en0, W_en1, b_en1, W_en2, b_en2, W_en3, b_en3,
           W_enfc, b_enfc, W_defc, b_defc,
           W_de0, b_de0, W_de1, b_de1, W_de2, b_de2, W_de3, b_de3,
           W_out, b_out):
    si = [si0, si1, si2, si3]
    down = [(d0_cols, d0_vals), (d1_cols, d1_vals),
            (d2_cols, d2_vals), (d3_cols, d3_vals)]
    up = [(u0_cols, u0_vals), (u1_cols, u1_vals),
          (u2_cols, u2_vals), (u3_cols, u3_vals)]
    blks = [512, 512, 768, 256, 256]

    b_sz = x.shape[0]
    xp = jnp.pad(x, ((0, 0), (0, 0), (0, 5)))
    w0p = jnp.pad(W_en0.reshape(W_en0.shape[0], SEQ, 3),
                  ((0, 0), (0, 0), (0, 5))).reshape(W_en0.shape[0], SEQ * 8)

    h = _spiral(xp, si0, NPAD[0], w0p, b_en0, True, blks[0])
    h = _pool(h, d0_cols, d0_vals, NPAD[1])
    h = _spiral(h, si1, NPAD[1], W_en1, b_en1, True, blks[1])
    h = _pool(h, d1_cols, d1_vals, NPAD[2])
    h = _spiral(h, si2, NPAD[2], W_en2, b_en2, True, blks[2])
    h = _pool(h, d2_cols, d2_vals, NPAD[3])
    h = _spiral(h, si3, NPAD[3], W_en3, b_en3, True, blks[3])
    h = _pool(h, d3_cols, d3_vals, NPAD[4])

    h_enc = h[:, :LEVELS[4]].reshape(b_sz, -1)
    mu, logvar, h2 = _mid(h_enc, W_enfc, b_enfc, eps, W_defc, b_defc)
    h = h2.reshape(b_sz, LEVELS[4], 64)

    h = _pool(h, u3_cols, u3_vals, NPAD[3])
    h = _spiral(h, si3, NPAD[3], W_de0, b_de0, True, blks[3])
    h = _pool(h, u2_cols, u2_vals, NPAD[2])
    h = _spiral(h, si2, NPAD[2], W_de1, b_de1, True, blks[2])
    h = _pool(h, u1_cols, u1_vals, NPAD[1])
    h = _spiral(h, si1, NPAD[1], W_de2, b_de2, True, blks[1])
    h = _pool(h, u0_cols, u0_vals, NPAD[0])
    h = _spiral(h, si0, NPAD[0], W_de3, b_de3, True, blks[0])

    re_x = _spiral(h, si0, NPAD[0], W_out, b_out, False, blks[0])
    return (re_x[:, :LEVELS[0]], mu, logvar)


# speedup vs baseline: 2.8504x; 1.2312x over previous
import functools

import jax
import jax.numpy as jnp
from jax import lax
from jax.experimental import pallas as pl
from jax.experimental.pallas import tpu as pltpu
from jax.experimental.pallas import tpu_sc as plsc

LEVELS = [10000, 2500, 625, 157, 40]
NPAD = [10240, 2560, 768, 256, 256]
SEQ = 9
B = 16
_NC, _NW = 2, 32
_MESH = dict(core_axis_name="c", subcore_axis_name="s")


def _sc_gather(h, idx):
    N, B_, C = h.shape
    R = B_ * C
    h2 = h.reshape(N, R)
    M = idx.shape[0]
    m_per = M // _NW
    CH = min(m_per, 49152 // R)

    @functools.partial(
        pl.kernel,
        out_type=jax.ShapeDtypeStruct((M, R), jnp.float32),
        mesh=plsc.VectorSubcoreMesh(**_MESH),
        compiler_params=pltpu.CompilerParams(use_tc_tiling_on_sc=False),
        scratch_types=[
            pltpu.VMEM((m_per,), jnp.int32),
            pltpu.VMEM((2, CH, R), jnp.float32),
            pltpu.SemaphoreType.DMA,
            pltpu.SemaphoreType.DMA,
        ],
    )
    def gk(h_hbm, idx_hbm, out_hbm, idx_v, buf_v, gsem, osem):
        wid = lax.axis_index("s") * _NC + lax.axis_index("c")
        base = wid * m_per
        pltpu.sync_copy(idx_hbm.at[pl.ds(base, m_per)], idx_v)

        prev_out = None
        for t, off in enumerate(range(0, m_per, CH)):
            n = min(CH, m_per - off)
            slot = t % 2
            cps = []
            o2 = 0
            while o2 < n:
                k = min(128, n - o2)
                cps.append(pltpu.async_copy(
                    h_hbm.at[idx_v.at[pl.ds(off + o2, k)]],
                    buf_v.at[slot].at[pl.ds(o2, k)], gsem))
                o2 += k
            for cp in cps:
                cp.wait()
            if prev_out is not None:
                prev_out.wait()
            prev_out = pltpu.async_copy(
                buf_v.at[slot].at[pl.ds(0, n)],
                out_hbm.at[pl.ds(base + off, n)], osem)
        prev_out.wait()

    return gk(h2, idx).reshape(M, B_, C)


def _sc_pool(h, cols, vals, nd):
    N, B_, C = h.shape
    R = B_ * C
    h2 = h.reshape(N, R)
    nd_per = nd // _NW
    CH = min(nd_per, 28672 // R)

    @functools.partial(
        pl.kernel,
        out_type=jax.ShapeDtypeStruct((nd, R), jnp.float32),
        mesh=plsc.VectorSubcoreMesh(**_MESH),
        compiler_params=pltpu.CompilerParams(use_tc_tiling_on_sc=False,
                                             needs_layout_passes=False),
        scratch_types=[
            pltpu.VMEM((3 * nd_per,), jnp.int32),
            pltpu.VMEM((3 * nd_per,), jnp.float32),
            pltpu.VMEM((3 * CH, R), jnp.float32),
            pltpu.VMEM((CH, R), jnp.float32),
            pltpu.SemaphoreType.DMA,
            pltpu.SemaphoreType.DMA,
        ],
    )
    def pk(h_hbm, cols_hbm, vals_hbm, out_hbm, idx_v, w_v, g_v, o_v,
           gsem, osem):
        wid = lax.axis_index("s") * _NC + lax.axis_index("c")
        base = wid * nd_per
        pltpu.sync_copy(cols_hbm.at[pl.ds(3 * base, 3 * nd_per)], idx_v)
        pltpu.sync_copy(vals_hbm.at[pl.ds(3 * base, 3 * nd_per)], w_v)

        prev_out = None
        for off in range(0, nd_per, CH):
            n = min(CH, nd_per - off)
            cps = []
            o2 = 0
            while o2 < 3 * n:
                k = min(128, 3 * n - o2)
                cps.append(pltpu.async_copy(
                    h_hbm.at[idx_v.at[pl.ds(3 * off + o2, k)]],
                    g_v.at[pl.ds(o2, k)], gsem))
                o2 += k
            for cp in cps:
                cp.wait()
            if prev_out is not None:
                prev_out.wait()

            def node(i, c2):
                e = 3 * off + 3 * i
                w0 = plsc.load_gather(w_v, [jnp.full((16,), e, jnp.int32)])
                w1 = plsc.load_gather(w_v, [jnp.full((16,), e + 1, jnp.int32)])
                w2 = plsc.load_gather(w_v, [jnp.full((16,), e + 2, jnp.int32)])
                for k in range(R // 16):
                    s = pl.ds(k * 16, 16)
                    o_v[i, s] = (w0 * g_v[3 * i, s] + w1 * g_v[3 * i + 1, s]
                                 + w2 * g_v[3 * i + 2, s])
                return c2

            lax.fori_loop(0, n, node, 0)
            prev_out = pltpu.async_copy(
                o_v.at[pl.ds(0, n)],
                out_hbm.at[pl.ds(base + off, n)], osem)
        prev_out.wait()

    return pk(h2, cols, vals).reshape(nd, B_, C)


def _conv_body(g_ref, w_ref, b_ref, o_ref, *, elu):
    blk, _, B_, C = g_ref.shape
    cout = o_ref.shape[2]
    acc = None
    for s in range(SEQ):
        gs = g_ref[:, s].reshape(blk * B_, C)
        ws = w_ref[:, s]
        p = lax.dot_general(gs, ws, (((1,), (1,)), ((), ())),
                            preferred_element_type=jnp.float32)
        acc = p if acc is None else acc + p
    y = acc + b_ref[...][None, :]
    if elu:
        y = jnp.where(y > 0, y, jnp.exp(jnp.minimum(y, 0.0)) - 1.0)
    o_ref[...] = y.reshape(blk, B_, cout)


def _conv(g, w, b, elu, blk):
    N, _, B_, C = g.shape
    cout = w.shape[0]
    return pl.pallas_call(
        functools.partial(_conv_body, elu=elu),
        grid=(N // blk,),
        in_specs=[
            pl.BlockSpec((blk, SEQ, B_, C), lambda j: (j, 0, 0, 0)),
            pl.BlockSpec((cout, SEQ, C), lambda j: (0, 0, 0)),
            pl.BlockSpec((cout,), lambda j: (0,)),
        ],
        out_specs=pl.BlockSpec((blk, B_, cout), lambda j: (j, 0, 0)),
        out_shape=jax.ShapeDtypeStruct((N, B_, cout), jnp.float32),
    )(g, w, b)


def _mid_body(h_ref, wen_ref, ben_ref, eps_ref, wde_ref, bde_ref,
              mu_ref, lv_ref, h2_ref):
    h = h_ref[...]
    y = lax.dot_general(h, wen_ref[...], (((1,), (1,)), ((), ())),
                        preferred_element_type=jnp.float32)
    y = y + ben_ref[...][None, :]
    mu = y[:, :64]
    logvar = y[:, 64:]
    z = mu + eps_ref[...] * jnp.exp(0.5 * logvar)
    h2 = lax.dot_general(z, wde_ref[...], (((1,), (1,)), ((), ())),
                         preferred_element_type=jnp.float32)
    h2 = h2 + bde_ref[...][None, :]
    mu_ref[...] = mu
    lv_ref[...] = logvar
    h2_ref[...] = h2


def _mid(h, w_enfc, b_enfc, eps, w_defc, b_defc):
    B_ = h.shape[0]
    d = w_defc.shape[0]
    return pl.pallas_call(
        _mid_body,
        out_shape=(
            jax.ShapeDtypeStruct((B_, 64), jnp.float32),
            jax.ShapeDtypeStruct((B_, 64), jnp.float32),
            jax.ShapeDtypeStruct((B_, d), jnp.float32),
        ),
    )(h, w_enfc, b_enfc, eps, w_defc, b_defc)


def _spiral(h, si, npad_dst, w, b, elu, blk):
    c = h.shape[2]
    cout = w.shape[0]
    flat = si.reshape(-1)
    m = npad_dst * SEQ
    if flat.shape[0] != m:
        flat = jnp.pad(flat, (0, m - flat.shape[0]))
    g = _sc_gather(h, flat)
    g = g.reshape(npad_dst, SEQ, h.shape[1], c)
    return _conv(g, w.reshape(cout, SEQ, c), b, elu, blk)


def _pool(h, cols, vals, npad_dst):
    e = 3 * npad_dst
    if cols.shape[0] != e:
        cols = jnp.pad(cols, (0, e - cols.shape[0]))
        vals = jnp.pad(vals, (0, e - vals.shape[0]))
    return _sc_pool(h, cols, vals, npad_dst)


def kernel(x, eps, si0, si1, si2, si3,
           d0_rows, d0_cols, d0_vals, d1_rows, d1_cols, d1_vals,
           d2_rows, d2_cols, d2_vals, d3_rows, d3_cols, d3_vals,
           u0_rows, u0_cols, u0_vals, u1_rows, u1_cols, u1_vals,
           u2_rows, u2_cols, u2_vals, u3_rows, u3_cols, u3_vals,
           W_en0, b_en0, W_en1, b_en1, W_en2, b_en2, W_en3, b_en3,
           W_enfc, b_enfc, W_defc, b_defc,
           W_de0, b_de0, W_de1, b_de1, W_de2, b_de2, W_de3, b_de3,
           W_out, b_out):
    b_sz = x.shape[0]

    xp = jnp.pad(x, ((0, 0), (0, 0), (0, 5))).transpose(1, 0, 2)
    w0p = jnp.pad(W_en0.reshape(W_en0.shape[0], SEQ, 3),
                  ((0, 0), (0, 0), (0, 5))).reshape(W_en0.shape[0], SEQ * 8)
    wop = jnp.pad(W_out, ((0, 5), (0, 0)))
    bop = jnp.pad(b_out, (0, 5))

    blks = [256, 256, 256, 256]

    h = _spiral(xp, si0, NPAD[0], w0p, b_en0, True, blks[0])
    h = _pool(h, d0_cols, d0_vals, NPAD[1])
    h = _spiral(h, si1, NPAD[1], W_en1, b_en1, True, blks[1])
    h = _pool(h, d1_cols, d1_vals, NPAD[2])
    h = _spiral(h, si2, NPAD[2], W_en2, b_en2, True, blks[2])
    h = _pool(h, d2_cols, d2_vals, NPAD[3])
    h = _spiral(h, si3, NPAD[3], W_en3, b_en3, True, blks[3])
    h = _pool(h, d3_cols, d3_vals, NPAD[4])

    h_enc = h[:LEVELS[4]].transpose(1, 0, 2).reshape(b_sz, -1)
    mu, logvar, h2 = _mid(h_enc, W_enfc, b_enfc, eps, W_defc, b_defc)
    h = h2.reshape(b_sz, LEVELS[4], 64).transpose(1, 0, 2)

    h = _pool(h, u3_cols, u3_vals, NPAD[3])
    h = _spiral(h, si3, NPAD[3], W_de0, b_de0, True, blks[3])
    h = _pool(h, u2_cols, u2_vals, NPAD[2])
    h = _spiral(h, si2, NPAD[2], W_de1, b_de1, True, blks[2])
    h = _pool(h, u1_cols, u1_vals, NPAD[1])
    h = _spiral(h, si1, NPAD[1], W_de2, b_de2, True, blks[1])
    h = _pool(h, u0_cols, u0_vals, NPAD[0])
    h = _spiral(h, si0, NPAD[0], W_de3, b_de3, True, blks[0])

    rx = _spiral(h, si0, NPAD[0], wop, bop, False, blks[0])
    re_x = rx[:LEVELS[0], :, :3].transpose(1, 0, 2)
    return (re_x, mu, logvar)


# speedup vs baseline: 3.1209x; 1.0949x over previous
import functools

import jax
import jax.numpy as jnp
from jax import lax
from jax.experimental import pallas as pl
from jax.experimental.pallas import tpu as pltpu
from jax.experimental.pallas import tpu_sc as plsc

LEVELS = [10000, 2500, 625, 157, 40]
NPAD = [10240, 2560, 768, 256, 256]
SEQ = 9
B = 16
_NC, _NW = 2, 32
_MESH = dict(core_axis_name="c", subcore_axis_name="s")


def _sc_gather(h, idx):
    N, B_, C = h.shape
    R = B_ * C
    h2 = h.reshape(N, R)
    M = idx.shape[0]
    m_per = M // _NW
    CH = min(m_per, 49152 // R)

    @functools.partial(
        pl.kernel,
        out_type=jax.ShapeDtypeStruct((M, R), jnp.float32),
        mesh=plsc.VectorSubcoreMesh(**_MESH),
        compiler_params=pltpu.CompilerParams(use_tc_tiling_on_sc=False),
        scratch_types=[
            pltpu.VMEM((m_per,), jnp.int32),
            pltpu.VMEM((2, CH, R), jnp.float32),
            pltpu.SemaphoreType.DMA,
            pltpu.SemaphoreType.DMA,
        ],
    )
    def gk(h_hbm, idx_hbm, out_hbm, idx_v, buf_v, gsem, osem):
        wid = lax.axis_index("s") * _NC + lax.axis_index("c")
        base = wid * m_per
        pltpu.sync_copy(idx_hbm.at[pl.ds(base, m_per)], idx_v)

        prev_out = None
        for t, off in enumerate(range(0, m_per, CH)):
            n = min(CH, m_per - off)
            slot = t % 2
            cps = []
            o2 = 0
            while o2 < n:
                k = min(128, n - o2)
                cps.append(pltpu.async_copy(
                    h_hbm.at[idx_v.at[pl.ds(off + o2, k)]],
                    buf_v.at[slot].at[pl.ds(o2, k)], gsem))
                o2 += k
            for cp in cps:
                cp.wait()
            if prev_out is not None:
                prev_out.wait()
            prev_out = pltpu.async_copy(
                buf_v.at[slot].at[pl.ds(0, n)],
                out_hbm.at[pl.ds(base + off, n)], osem)
        prev_out.wait()

    return gk(h2, idx).reshape(M, B_, C)


def _sc_pool(h, cols, vals, nd):
    N, B_, C = h.shape
    R = B_ * C
    h2 = h.reshape(N, R)
    nd_per = nd // _NW
    CH = min(nd_per, 28672 // R)

    @functools.partial(
        pl.kernel,
        out_type=jax.ShapeDtypeStruct((nd, R), jnp.float32),
        mesh=plsc.VectorSubcoreMesh(**_MESH),
        compiler_params=pltpu.CompilerParams(use_tc_tiling_on_sc=False,
                                             needs_layout_passes=False),
        scratch_types=[
            pltpu.VMEM((3 * nd_per,), jnp.int32),
            pltpu.VMEM((3 * nd_per,), jnp.float32),
            pltpu.VMEM((3 * CH, R), jnp.float32),
            pltpu.VMEM((CH, R), jnp.float32),
            pltpu.SemaphoreType.DMA,
            pltpu.SemaphoreType.DMA,
        ],
    )
    def pk(h_hbm, cols_hbm, vals_hbm, out_hbm, idx_v, w_v, g_v, o_v,
           gsem, osem):
        wid = lax.axis_index("s") * _NC + lax.axis_index("c")
        base = wid * nd_per
        pltpu.sync_copy(cols_hbm.at[pl.ds(3 * base, 3 * nd_per)], idx_v)
        pltpu.sync_copy(vals_hbm.at[pl.ds(3 * base, 3 * nd_per)], w_v)

        prev_out = None
        for off in range(0, nd_per, CH):
            n = min(CH, nd_per - off)
            cps = []
            o2 = 0
            while o2 < 3 * n:
                k = min(128, 3 * n - o2)
                cps.append(pltpu.async_copy(
                    h_hbm.at[idx_v.at[pl.ds(3 * off + o2, k)]],
                    g_v.at[pl.ds(o2, k)], gsem))
                o2 += k
            for cp in cps:
                cp.wait()
            if prev_out is not None:
                prev_out.wait()

            def node(i, c2):
                e = 3 * off + 3 * i
                w0 = plsc.load_gather(w_v, [jnp.full((16,), e, jnp.int32)])
                w1 = plsc.load_gather(w_v, [jnp.full((16,), e + 1, jnp.int32)])
                w2 = plsc.load_gather(w_v, [jnp.full((16,), e + 2, jnp.int32)])
                for k in range(R // 16):
                    s = pl.ds(k * 16, 16)
                    o_v[i, s] = (w0 * g_v[3 * i, s] + w1 * g_v[3 * i + 1, s]
                                 + w2 * g_v[3 * i + 2, s])
                return c2

            lax.fori_loop(0, n, node, 0)
            prev_out = pltpu.async_copy(
                o_v.at[pl.ds(0, n)],
                out_hbm.at[pl.ds(base + off, n)], osem)
        prev_out.wait()

    return pk(h2, cols, vals).reshape(nd, B_, C)


def _sc_gacc(p, fidx, b16, nd):
    M9, R = p.shape
    nd_per = nd // _NW
    CH = min(nd_per, (11468 // R) & ~7)

    @functools.partial(
        pl.kernel,
        out_type=jax.ShapeDtypeStruct((nd, R), jnp.float32),
        mesh=plsc.VectorSubcoreMesh(**_MESH),
        compiler_params=pltpu.CompilerParams(use_tc_tiling_on_sc=False,
                                             needs_layout_passes=False),
        scratch_types=[
            pltpu.VMEM((SEQ * nd_per,), jnp.int32),
            pltpu.VMEM((16,), jnp.float32),
            pltpu.VMEM((SEQ * CH, R), jnp.float32),
            pltpu.VMEM((CH, R), jnp.float32),
            pltpu.SemaphoreType.DMA,
            pltpu.SemaphoreType.DMA,
        ],
    )
    def ak(p_hbm, fidx_hbm, b_hbm, out_hbm, idx_v, b_v, g_v, o_v,
           gsem, osem):
        wid = lax.axis_index("s") * _NC + lax.axis_index("c")
        base = wid * nd_per
        pltpu.sync_copy(fidx_hbm.at[pl.ds(SEQ * base, SEQ * nd_per)], idx_v)
        pltpu.sync_copy(b_hbm, b_v)
        bvec = b_v[pl.ds(0, 16)]

        prev_out = None
        for off in range(0, nd_per, CH):
            n = min(CH, nd_per - off)
            cps = []
            o2 = 0
            while o2 < SEQ * n:
                k = min(128, SEQ * n - o2)
                cps.append(pltpu.async_copy(
                    p_hbm.at[idx_v.at[pl.ds(SEQ * off + o2, k)]],
                    g_v.at[pl.ds(o2, k)], gsem))
                o2 += k
            for cp in cps:
                cp.wait()
            if prev_out is not None:
                prev_out.wait()

            def node(i, c2):
                for k in range(R // 16):
                    s = pl.ds(k * 16, 16)
                    acc = bvec + g_v[SEQ * i, s]
                    for j in range(1, SEQ):
                        acc = acc + g_v[SEQ * i + j, s]
                    o_v[i, s] = acc
                return c2

            lax.fori_loop(0, n, node, 0)
            prev_out = pltpu.async_copy(
                o_v.at[pl.ds(0, n)],
                out_hbm.at[pl.ds(base + off, n)], osem)
        prev_out.wait()

    return ak(p, fidx, b16)


def _conv_body(g_ref, w_ref, b_ref, o_ref, *, elu):
    blk, _, B_, C = g_ref.shape
    cout = o_ref.shape[2]
    acc = None
    for s in range(SEQ):
        gs = g_ref[:, s].reshape(blk * B_, C)
        ws = w_ref[:, s]
        p = lax.dot_general(gs, ws, (((1,), (1,)), ((), ())),
                            preferred_element_type=jnp.float32)
        acc = p if acc is None else acc + p
    y = acc + b_ref[...][None, :]
    if elu:
        y = jnp.where(y > 0, y, jnp.exp(jnp.minimum(y, 0.0)) - 1.0)
    o_ref[...] = y.reshape(blk, B_, cout)


def _conv(g, w, b, elu, blk):
    N, _, B_, C = g.shape
    cout = w.shape[0]
    return pl.pallas_call(
        functools.partial(_conv_body, elu=elu),
        grid=(N // blk,),
        in_specs=[
            pl.BlockSpec((blk, SEQ, B_, C), lambda j: (j, 0, 0, 0)),
            pl.BlockSpec((cout, SEQ, C), lambda j: (0, 0, 0)),
            pl.BlockSpec((cout,), lambda j: (0,)),
        ],
        out_specs=pl.BlockSpec((blk, B_, cout), lambda j: (j, 0, 0)),
        out_shape=jax.ShapeDtypeStruct((N, B_, cout), jnp.float32),
    )(g, w, b)


def _conv_partials_body(g_ref, w_ref, b_ref, w2_ref, o_ref):
    blk, _, B_, C = g_ref.shape
    cout2 = o_ref.shape[3]
    acc = None
    for s in range(SEQ):
        gs = g_ref[:, s].reshape(blk * B_, C)
        p = lax.dot_general(gs, w_ref[:, s], (((1,), (1,)), ((), ())),
                            preferred_element_type=jnp.float32)
        acc = p if acc is None else acc + p
    cout = acc.shape[1]
    y = acc + b_ref[...][None, :]
    y = jnp.where(y > 0, y, jnp.exp(jnp.minimum(y, 0.0)) - 1.0)
    for s in range(SEQ):
        ps = lax.dot_general(y, w2_ref[:, s], (((1,), (1,)), ((), ())),
                             preferred_element_type=jnp.float32)
        o_ref[:, s] = ps.reshape(blk, B_, cout2)


def _conv_partials(g, w, b, w2, blk):
    N, _, B_, C = g.shape
    cout = w.shape[0]
    cout2 = w2.shape[0]
    return pl.pallas_call(
        _conv_partials_body,
        grid=(N // blk,),
        in_specs=[
            pl.BlockSpec((blk, SEQ, B_, C), lambda j: (j, 0, 0, 0)),
            pl.BlockSpec((cout, SEQ, C), lambda j: (0, 0, 0)),
            pl.BlockSpec((cout,), lambda j: (0,)),
            pl.BlockSpec((cout2, SEQ, cout), lambda j: (0, 0, 0)),
        ],
        out_specs=pl.BlockSpec((blk, SEQ, B_, cout2), lambda j: (j, 0, 0, 0)),
        out_shape=jax.ShapeDtypeStruct((N, SEQ, B_, cout2), jnp.float32),
    )(g, w.reshape(cout, SEQ, C), b, w2.reshape(cout2, SEQ, cout))


def _mid_body(h_ref, wen_ref, ben_ref, eps_ref, wde_ref, bde_ref,
              mu_ref, lv_ref, h2_ref):
    h = h_ref[...]
    y = lax.dot_general(h, wen_ref[...], (((1,), (1,)), ((), ())),
                        preferred_element_type=jnp.float32)
    y = y + ben_ref[...][None, :]
    mu = y[:, :64]
    logvar = y[:, 64:]
    z = mu + eps_ref[...] * jnp.exp(0.5 * logvar)
    h2 = lax.dot_general(z, wde_ref[...], (((1,), (1,)), ((), ())),
                         preferred_element_type=jnp.float32)
    h2 = h2 + bde_ref[...][None, :]
    mu_ref[...] = mu
    lv_ref[...] = logvar
    h2_ref[...] = h2


def _mid(h, w_enfc, b_enfc, eps, w_defc, b_defc):
    B_ = h.shape[0]
    d = w_defc.shape[0]
    return pl.pallas_call(
        _mid_body,
        out_shape=(
            jax.ShapeDtypeStruct((B_, 64), jnp.float32),
            jax.ShapeDtypeStruct((B_, 64), jnp.float32),
            jax.ShapeDtypeStruct((B_, d), jnp.float32),
        ),
    )(h, w_enfc, b_enfc, eps, w_defc, b_defc)


def _spiral(h, si, npad_dst, w, b, elu, blk):
    c = h.shape[2]
    cout = w.shape[0]
    flat = si.reshape(-1)
    m = npad_dst * SEQ
    if flat.shape[0] != m:
        flat = jnp.pad(flat, (0, m - flat.shape[0]))
    g = _sc_gather(h, flat)
    g = g.reshape(npad_dst, SEQ, h.shape[1], c)
    return _conv(g, w.reshape(cout, SEQ, c), b, elu, blk)


def _pool(h, cols, vals, npad_dst):
    e = 3 * npad_dst
    if cols.shape[0] != e:
        cols = jnp.pad(cols, (0, e - cols.shape[0]))
        vals = jnp.pad(vals, (0, e - vals.shape[0]))
    return _sc_pool(h, cols, vals, npad_dst)


def kernel(x, eps, si0, si1, si2, si3,
           d0_rows, d0_cols, d0_vals, d1_rows, d1_cols, d1_vals,
           d2_rows, d2_cols, d2_vals, d3_rows, d3_cols, d3_vals,
           u0_rows, u0_cols, u0_vals, u1_rows, u1_cols, u1_vals,
           u2_rows, u2_cols, u2_vals, u3_rows, u3_cols, u3_vals,
           W_en0, b_en0, W_en1, b_en1, W_en2, b_en2, W_en3, b_en3,
           W_enfc, b_enfc, W_defc, b_defc,
           W_de0, b_de0, W_de1, b_de1, W_de2, b_de2, W_de3, b_de3,
           W_out, b_out):
    b_sz = x.shape[0]

    xp = jnp.pad(x, ((0, 0), (0, 0), (0, 5))).transpose(1, 0, 2)
    w0p = jnp.pad(W_en0.reshape(W_en0.shape[0], SEQ, 3),
                  ((0, 0), (0, 0), (0, 5))).reshape(W_en0.shape[0], SEQ * 8)
    wop = jnp.pad(W_out, ((0, 5), (0, 0)))
    bop = jnp.pad(b_out, (0, 5))

    blks = [256, 256, 256, 256]

    h = _spiral(xp, si0, NPAD[0], w0p, b_en0, True, blks[0])
    h = _pool(h, d0_cols, d0_vals, NPAD[1])
    h = _spiral(h, si1, NPAD[1], W_en1, b_en1, True, blks[1])
    h = _pool(h, d1_cols, d1_vals, NPAD[2])
    h = _spiral(h, si2, NPAD[2], W_en2, b_en2, True, blks[2])
    h = _pool(h, d2_cols, d2_vals, NPAD[3])
    h = _spiral(h, si3, NPAD[3], W_en3, b_en3, True, blks[3])
    h = _pool(h, d3_cols, d3_vals, NPAD[4])

    h_enc = h[:LEVELS[4]].transpose(1, 0, 2).reshape(b_sz, -1)
    mu, logvar, h2 = _mid(h_enc, W_enfc, b_enfc, eps, W_defc, b_defc)
    h = h2.reshape(b_sz, LEVELS[4], 64).transpose(1, 0, 2)

    h = _pool(h, u3_cols, u3_vals, NPAD[3])
    h = _spiral(h, si3, NPAD[3], W_de0, b_de0, True, blks[3])
    h = _pool(h, u2_cols, u2_vals, NPAD[2])
    h = _spiral(h, si2, NPAD[2], W_de1, b_de1, True, blks[2])
    h = _pool(h, u1_cols, u1_vals, NPAD[1])
    h = _spiral(h, si1, NPAD[1], W_de2, b_de2, True, blks[1])
    h = _pool(h, u0_cols, u0_vals, NPAD[0])

    flat0 = si0.reshape(-1)
    m0 = NPAD[0] * SEQ
    if flat0.shape[0] != m0:
        flat0 = jnp.pad(flat0, (0, m0 - flat0.shape[0]))
    g = _sc_gather(h, flat0)
    g = g.reshape(NPAD[0], SEQ, B, 32)
    p = _conv_partials(g, W_de3, b_de3, wop, 64)
    fidx = (flat0.reshape(-1, SEQ) * SEQ
            + jnp.arange(SEQ, dtype=jnp.int32)[None, :]).reshape(-1)
    b16 = jnp.tile(bop, 2)
    rx = _sc_gacc(p.reshape(m0, B * 8), fidx, b16, NPAD[0])
    rx = rx.reshape(NPAD[0], B, 8)
    re_x = rx[:LEVELS[0], :, :3].transpose(1, 0, 2)
    return (re_x, mu, logvar)


# speedup vs baseline: 3.1662x; 1.0145x over previous
import functools

import jax
import jax.numpy as jnp
from jax import lax
from jax.experimental import pallas as pl
from jax.experimental.pallas import tpu as pltpu
from jax.experimental.pallas import tpu_sc as plsc

LEVELS = [10000, 2500, 625, 157, 40]
NPAD = [10240, 2560, 768, 256, 256]
SEQ = 9
B = 16
_NC, _NW = 2, 32
_MESH = dict(core_axis_name="c", subcore_axis_name="s")


def _sc_gather(h, idx):
    N, B_, C = h.shape
    R = B_ * C
    h2 = h.reshape(N, R)
    M = idx.shape[0]
    m_per = M // _NW
    CH = min(m_per, 49152 // R)

    @functools.partial(
        pl.kernel,
        out_type=jax.ShapeDtypeStruct((M, R), jnp.float32),
        mesh=plsc.VectorSubcoreMesh(**_MESH),
        compiler_params=pltpu.CompilerParams(use_tc_tiling_on_sc=False),
        scratch_types=[
            pltpu.VMEM((m_per,), jnp.int32),
            pltpu.VMEM((2, CH, R), jnp.float32),
            pltpu.SemaphoreType.DMA,
            pltpu.SemaphoreType.DMA,
        ],
    )
    def gk(h_hbm, idx_hbm, out_hbm, idx_v, buf_v, gsem, osem):
        wid = lax.axis_index("s") * _NC + lax.axis_index("c")
        base = wid * m_per
        pltpu.sync_copy(idx_hbm.at[pl.ds(base, m_per)], idx_v)

        prev_out = None
        for t, off in enumerate(range(0, m_per, CH)):
            n = min(CH, m_per - off)
            slot = t % 2
            cps = []
            o2 = 0
            while o2 < n:
                k = min(512, n - o2)
                cps.append(pltpu.async_copy(
                    h_hbm.at[idx_v.at[pl.ds(off + o2, k)]],
                    buf_v.at[slot].at[pl.ds(o2, k)], gsem))
                o2 += k
            for cp in cps:
                cp.wait()
            if prev_out is not None:
                prev_out.wait()
            prev_out = pltpu.async_copy(
                buf_v.at[slot].at[pl.ds(0, n)],
                out_hbm.at[pl.ds(base + off, n)], osem)
        prev_out.wait()

    return gk(h2, idx).reshape(M, B_, C)


def _sc_pool(h, cols, vals, nd):
    N, B_, C = h.shape
    R = B_ * C
    h2 = h.reshape(N, R)
    nd_per = nd // _NW
    CH = min(nd_per, 28672 // R)

    @functools.partial(
        pl.kernel,
        out_type=jax.ShapeDtypeStruct((nd, R), jnp.float32),
        mesh=plsc.VectorSubcoreMesh(**_MESH),
        compiler_params=pltpu.CompilerParams(use_tc_tiling_on_sc=False,
                                             needs_layout_passes=False),
        scratch_types=[
            pltpu.VMEM((3 * nd_per,), jnp.int32),
            pltpu.VMEM((3 * nd_per,), jnp.float32),
            pltpu.VMEM((3 * CH, R), jnp.float32),
            pltpu.VMEM((CH, R), jnp.float32),
            pltpu.SemaphoreType.DMA,
            pltpu.SemaphoreType.DMA,
        ],
    )
    def pk(h_hbm, cols_hbm, vals_hbm, out_hbm, idx_v, w_v, g_v, o_v,
           gsem, osem):
        wid = lax.axis_index("s") * _NC + lax.axis_index("c")
        base = wid * nd_per
        pltpu.sync_copy(cols_hbm.at[pl.ds(3 * base, 3 * nd_per)], idx_v)
        pltpu.sync_copy(vals_hbm.at[pl.ds(3 * base, 3 * nd_per)], w_v)

        prev_out = None
        for off in range(0, nd_per, CH):
            n = min(CH, nd_per - off)
            cps = []
            o2 = 0
            while o2 < 3 * n:
                k = min(512, 3 * n - o2)
                cps.append(pltpu.async_copy(
                    h_hbm.at[idx_v.at[pl.ds(3 * off + o2, k)]],
                    g_v.at[pl.ds(o2, k)], gsem))
                o2 += k
            for cp in cps:
                cp.wait()
            if prev_out is not None:
                prev_out.wait()

            def node(i, c2):
                e = 3 * off + 3 * i
                w0 = plsc.load_gather(w_v, [jnp.full((16,), e, jnp.int32)])
                w1 = plsc.load_gather(w_v, [jnp.full((16,), e + 1, jnp.int32)])
                w2 = plsc.load_gather(w_v, [jnp.full((16,), e + 2, jnp.int32)])
                for k in range(R // 16):
                    s = pl.ds(k * 16, 16)
                    o_v[i, s] = (w0 * g_v[3 * i, s] + w1 * g_v[3 * i + 1, s]
                                 + w2 * g_v[3 * i + 2, s])
                return c2

            lax.fori_loop(0, n, node, 0)
            prev_out = pltpu.async_copy(
                o_v.at[pl.ds(0, n)],
                out_hbm.at[pl.ds(base + off, n)], osem)
        prev_out.wait()

    return pk(h2, cols, vals).reshape(nd, B_, C)


def _sc_gacc(p, fidx, b16, nd):
    M9, R = p.shape
    nd_per = nd // _NW
    CH = min(nd_per, (11468 // R) & ~7)

    @functools.partial(
        pl.kernel,
        out_type=jax.ShapeDtypeStruct((nd, R), jnp.float32),
        mesh=plsc.VectorSubcoreMesh(**_MESH),
        compiler_params=pltpu.CompilerParams(use_tc_tiling_on_sc=False,
                                             needs_layout_passes=False),
        scratch_types=[
            pltpu.VMEM((SEQ * nd_per,), jnp.int32),
            pltpu.VMEM((16,), jnp.float32),
            pltpu.VMEM((SEQ * CH, R), jnp.float32),
            pltpu.VMEM((CH, R), jnp.float32),
            pltpu.SemaphoreType.DMA,
            pltpu.SemaphoreType.DMA,
        ],
    )
    def ak(p_hbm, fidx_hbm, b_hbm, out_hbm, idx_v, b_v, g_v, o_v,
           gsem, osem):
        wid = lax.axis_index("s") * _NC + lax.axis_index("c")
        base = wid * nd_per
        pltpu.sync_copy(fidx_hbm.at[pl.ds(SEQ * base, SEQ * nd_per)], idx_v)
        pltpu.sync_copy(b_hbm, b_v)
        bvec = b_v[pl.ds(0, 16)]

        prev_out = None
        for off in range(0, nd_per, CH):
            n = min(CH, nd_per - off)
            cps = []
            o2 = 0
            while o2 < SEQ * n:
                k = min(512, SEQ * n - o2)
                cps.append(pltpu.async_copy(
                    p_hbm.at[idx_v.at[pl.ds(SEQ * off + o2, k)]],
                    g_v.at[pl.ds(o2, k)], gsem))
                o2 += k
            for cp in cps:
                cp.wait()
            if prev_out is not None:
                prev_out.wait()

            def node(i, c2):
                for k in range(R // 16):
                    s = pl.ds(k * 16, 16)
                    acc = bvec + g_v[SEQ * i, s]
                    for j in range(1, SEQ):
                        acc = acc + g_v[SEQ * i + j, s]
                    o_v[i, s] = acc
                return c2

            lax.fori_loop(0, n, node, 0)
            prev_out = pltpu.async_copy(
                o_v.at[pl.ds(0, n)],
                out_hbm.at[pl.ds(base + off, n)], osem)
        prev_out.wait()

    return ak(p, fidx, b16)


def _conv_body(g_ref, w_ref, b_ref, o_ref, *, elu):
    blk, _, B_, C = g_ref.shape
    cout = o_ref.shape[2]
    acc = None
    for s in range(SEQ):
        gs = g_ref[:, s].reshape(blk * B_, C)
        ws = w_ref[:, s]
        p = lax.dot_general(gs, ws, (((1,), (1,)), ((), ())),
                            preferred_element_type=jnp.float32)
        acc = p if acc is None else acc + p
    y = acc + b_ref[...][None, :]
    if elu:
        y = jnp.where(y > 0, y, jnp.exp(jnp.minimum(y, 0.0)) - 1.0)
    o_ref[...] = y.reshape(blk, B_, cout)


def _conv(g, w, b, elu, blk):
    N, _, B_, C = g.shape
    cout = w.shape[0]
    return pl.pallas_call(
        functools.partial(_conv_body, elu=elu),
        grid=(N // blk,),
        in_specs=[
            pl.BlockSpec((blk, SEQ, B_, C), lambda j: (j, 0, 0, 0)),
            pl.BlockSpec((cout, SEQ, C), lambda j: (0, 0, 0)),
            pl.BlockSpec((cout,), lambda j: (0,)),
        ],
        out_specs=pl.BlockSpec((blk, B_, cout), lambda j: (j, 0, 0)),
        out_shape=jax.ShapeDtypeStruct((N, B_, cout), jnp.float32),
    )(g, w, b)


def _conv_partials_body(g_ref, w_ref, b_ref, w2_ref, o_ref):
    blk, _, B_, C = g_ref.shape
    cout2 = o_ref.shape[3]
    acc = None
    for s in range(SEQ):
        gs = g_ref[:, s].reshape(blk * B_, C)
        p = lax.dot_general(gs, w_ref[:, s], (((1,), (1,)), ((), ())),
                            preferred_element_type=jnp.float32)
        acc = p if acc is None else acc + p
    cout = acc.shape[1]
    y = acc + b_ref[...][None, :]
    y = jnp.where(y > 0, y, jnp.exp(jnp.minimum(y, 0.0)) - 1.0)
    for s in range(SEQ):
        ps = lax.dot_general(y, w2_ref[:, s], (((1,), (1,)), ((), ())),
                             preferred_element_type=jnp.float32)
        o_ref[:, s] = ps.reshape(blk, B_, cout2)


def _conv_partials(g, w, b, w2, blk):
    N, _, B_, C = g.shape
    cout = w.shape[0]
    cout2 = w2.shape[0]
    return pl.pallas_call(
        _conv_partials_body,
        grid=(N // blk,),
        in_specs=[
            pl.BlockSpec((blk, SEQ, B_, C), lambda j: (j, 0, 0, 0)),
            pl.BlockSpec((cout, SEQ, C), lambda j: (0, 0, 0)),
            pl.BlockSpec((cout,), lambda j: (0,)),
            pl.BlockSpec((cout2, SEQ, cout), lambda j: (0, 0, 0)),
        ],
        out_specs=pl.BlockSpec((blk, SEQ, B_, cout2), lambda j: (j, 0, 0, 0)),
        out_shape=jax.ShapeDtypeStruct((N, SEQ, B_, cout2), jnp.float32),
    )(g, w.reshape(cout, SEQ, C), b, w2.reshape(cout2, SEQ, cout))


def _mid_body(h_ref, wen_ref, ben_ref, eps_ref, wde_ref, bde_ref,
              mu_ref, lv_ref, h2_ref):
    h = h_ref[...]
    y = lax.dot_general(h, wen_ref[...], (((1,), (1,)), ((), ())),
                        preferred_element_type=jnp.float32)
    y = y + ben_ref[...][None, :]
    mu = y[:, :64]
    logvar = y[:, 64:]
    z = mu + eps_ref[...] * jnp.exp(0.5 * logvar)
    h2 = lax.dot_general(z, wde_ref[...], (((1,), (1,)), ((), ())),
                         preferred_element_type=jnp.float32)
    h2 = h2 + bde_ref[...][None, :]
    mu_ref[...] = mu
    lv_ref[...] = logvar
    h2_ref[...] = h2


def _mid(h, w_enfc, b_enfc, eps, w_defc, b_defc):
    B_ = h.shape[0]
    d = w_defc.shape[0]
    return pl.pallas_call(
        _mid_body,
        out_shape=(
            jax.ShapeDtypeStruct((B_, 64), jnp.float32),
            jax.ShapeDtypeStruct((B_, 64), jnp.float32),
            jax.ShapeDtypeStruct((B_, d), jnp.float32),
        ),
    )(h, w_enfc, b_enfc, eps, w_defc, b_defc)


def _spiral(h, si, npad_dst, w, b, elu, blk):
    c = h.shape[2]
    cout = w.shape[0]
    flat = si.reshape(-1)
    m = npad_dst * SEQ
    if flat.shape[0] != m:
        flat = jnp.pad(flat, (0, m - flat.shape[0]))
    g = _sc_gather(h, flat)
    g = g.reshape(npad_dst, SEQ, h.shape[1], c)
    return _conv(g, w.reshape(cout, SEQ, c), b, elu, blk)


def _pool(h, cols, vals, npad_dst):
    e = 3 * npad_dst
    if cols.shape[0] != e:
        cols = jnp.pad(cols, (0, e - cols.shape[0]))
        vals = jnp.pad(vals, (0, e - vals.shape[0]))
    return _sc_pool(h, cols, vals, npad_dst)


def kernel(x, eps, si0, si1, si2, si3,
           d0_rows, d0_cols, d0_vals, d1_rows, d1_cols, d1_vals,
           d2_rows, d2_cols, d2_vals, d3_rows, d3_cols, d3_vals,
           u0_rows, u0_cols, u0_vals, u1_rows, u1_cols, u1_vals,
           u2_rows, u2_cols, u2_vals, u3_rows, u3_cols, u3_vals,
           W_en0, b_en0, W_en1, b_en1, W_en2, b_en2, W_en3, b_en3,
           W_enfc, b_enfc, W_defc, b_defc,
           W_de0, b_de0, W_de1, b_de1, W_de2, b_de2, W_de3, b_de3,
           W_out, b_out):
    b_sz = x.shape[0]

    xp = jnp.pad(x, ((0, 0), (0, 0), (0, 5))).transpose(1, 0, 2)
    w0p = jnp.pad(W_en0.reshape(W_en0.shape[0], SEQ, 3),
                  ((0, 0), (0, 0), (0, 5))).reshape(W_en0.shape[0], SEQ * 8)
    wop = jnp.pad(W_out, ((0, 5), (0, 0)))
    bop = jnp.pad(b_out, (0, 5))

    blks = [256, 256, 256, 256]

    h = _spiral(xp, si0, NPAD[0], w0p, b_en0, True, blks[0])
    h = _pool(h, d0_cols, d0_vals, NPAD[1])
    h = _spiral(h, si1, NPAD[1], W_en1, b_en1, True, blks[1])
    h = _pool(h, d1_cols, d1_vals, NPAD[2])
    h = _spiral(h, si2, NPAD[2], W_en2, b_en2, True, blks[2])
    h = _pool(h, d2_cols, d2_vals, NPAD[3])
    h = _spiral(h, si3, NPAD[3], W_en3, b_en3, True, blks[3])
    h = _pool(h, d3_cols, d3_vals, NPAD[4])

    h_enc = h[:LEVELS[4]].transpose(1, 0, 2).reshape(b_sz, -1)
    mu, logvar, h2 = _mid(h_enc, W_enfc, b_enfc, eps, W_defc, b_defc)
    h = h2.reshape(b_sz, LEVELS[4], 64).transpose(1, 0, 2)

    h = _pool(h, u3_cols, u3_vals, NPAD[3])
    h = _spiral(h, si3, NPAD[3], W_de0, b_de0, True, blks[3])
    h = _pool(h, u2_cols, u2_vals, NPAD[2])
    h = _spiral(h, si2, NPAD[2], W_de1, b_de1, True, blks[2])
    h = _pool(h, u1_cols, u1_vals, NPAD[1])
    h = _spiral(h, si1, NPAD[1], W_de2, b_de2, True, blks[1])
    h = _pool(h, u0_cols, u0_vals, NPAD[0])

    flat0 = si0.reshape(-1)
    m0 = NPAD[0] * SEQ
    if flat0.shape[0] != m0:
        flat0 = jnp.pad(flat0, (0, m0 - flat0.shape[0]))
    g = _sc_gather(h, flat0)
    g = g.reshape(NPAD[0], SEQ, B, 32)
    p = _conv_partials(g, W_de3, b_de3, wop, 128)
    fidx = (flat0.reshape(-1, SEQ) * SEQ
            + jnp.arange(SEQ, dtype=jnp.int32)[None, :]).reshape(-1)
    b16 = jnp.tile(bop, 2)
    rx = _sc_gacc(p.reshape(m0, B * 8), fidx, b16, NPAD[0])
    rx = rx.reshape(NPAD[0], B, 8)
    re_x = rx[:LEVELS[0], :, :3].transpose(1, 0, 2)
    return (re_x, mu, logvar)
